# Initial kernel scaffold; baseline (speedup 1.0000x reference)
import dataclasses
import functools

import jax
import jax.numpy as jnp
from jax import lax
from jax.experimental import pallas as pl
from jax.experimental.pallas import tpu as pltpu
from jax.experimental.pallas import tpu_sc as plsc

N = 10000
F = 128
E = 320000
NC = 2
NS = 16
NW = NC * NS
EPW = E // NW
C = 80
NCH = EPW // C
NPAD = 10240
STRIPE = NPAD // NS


def _sc_compiler_params():
    cp = pltpu.CompilerParams()
    if "needs_layout_passes" in pltpu.CompilerParams.__dataclass_fields__:
        cp = dataclasses.replace(cp, needs_layout_passes=False)
    return cp


def _dense_in_body(x_ref, w_ref, a_ref, h_ref, esd_ref):
    h = jnp.dot(x_ref[...], w_ref[...], preferred_element_type=jnp.float32,
                precision=lax.Precision.HIGHEST)
    h_ref[...] = h
    es = jnp.sum(h * a_ref[0][None, :], axis=1, keepdims=True)
    ed = jnp.sum(h * a_ref[1][None, :], axis=1, keepdims=True)
    esd_ref[...] = jnp.concatenate(
        [es, ed, jnp.zeros((N, F - 2), jnp.float32)], axis=1)


def _dense_in(x, W, a2):
    return pl.pallas_call(
        _dense_in_body,
        out_shape=[jax.ShapeDtypeStruct((N, F), jnp.float32),
                   jax.ShapeDtypeStruct((N, F), jnp.float32)],
    )(x, W, a2)


def _dense_mid_body(acc_ref, den_ref, w_ref, a_ref, h_ref, esd_ref):
    d = den_ref[0, :N] + den_ref[1, :N]
    y = (acc_ref[0, :N] + acc_ref[1, :N]) / (d + 1e-16)
    y = jnp.maximum(y, 0.0)
    h = jnp.dot(y, w_ref[...], preferred_element_type=jnp.float32,
                precision=lax.Precision.HIGHEST)
    h_ref[...] = h
    es = jnp.sum(h * a_ref[0][None, :], axis=1, keepdims=True)
    ed = jnp.sum(h * a_ref[1][None, :], axis=1, keepdims=True)
    esd_ref[...] = jnp.concatenate(
        [es, ed, jnp.zeros((N, F - 2), jnp.float32)], axis=1)


def _dense_mid(acc, den, W, a2):
    return pl.pallas_call(
        _dense_mid_body,
        out_shape=[jax.ShapeDtypeStruct((N, F), jnp.float32),
                   jax.ShapeDtypeStruct((N, F), jnp.float32)],
    )(acc, den, W, a2)


def _dense_out_body(acc_ref, den_ref, o_ref):
    d = den_ref[0, :N] + den_ref[1, :N]
    o_ref[...] = (acc_ref[0, :N] + acc_ref[1, :N]) / (d + 1e-16)


def _dense_out(acc, den):
    return pl.pallas_call(
        _dense_out_body,
        out_shape=jax.ShapeDtypeStruct((N, F), jnp.float32),
    )(acc, den)


def _edge_sc(h, esd, src3, dst3):
    mesh = plsc.VectorSubcoreMesh(core_axis_name="c", subcore_axis_name="s")

    @functools.partial(
        pl.kernel,
        out_type=[jax.ShapeDtypeStruct((NC, NPAD, F), jnp.float32),
                  jax.ShapeDtypeStruct((NC, NPAD), jnp.float32)],
        mesh=mesh,
        scratch_types=[
            pltpu.VMEM((N,), jnp.float32),
            pltpu.VMEM((N,), jnp.float32),
            pltpu.VMEM((NCH, C), jnp.int32),
            pltpu.VMEM((NCH, C), jnp.int32),
            pltpu.VMEM((NCH, C), jnp.float32),
            pltpu.VMEM((C, F), jnp.float32),
            pltpu.VMEM((STRIPE,), jnp.float32),
            pltpu.VMEM_SHARED((NPAD, F), jnp.float32),
            pltpu.VMEM_SHARED((NPAD,), jnp.float32),
        ],
        compiler_params=_sc_compiler_params(),
    )
    def k(h_hbm, esd_hbm, src_hbm, dst_hbm, acc_hbm, den_hbm,
          es_v, ed_v, src_v, dst_v, ex_v, rows_v, zvec_v, acc_sh, den_sh):
        cid = lax.axis_index("c")
        sid = lax.axis_index("s")
        wid = sid * NC + cid

        pltpu.sync_copy(esd_hbm.at[0], es_v)
        pltpu.sync_copy(esd_hbm.at[1], ed_v)
        pltpu.sync_copy(src_hbm.at[wid], src_v)
        pltpu.sync_copy(dst_hbm.at[wid], dst_v)

        @pl.loop(0, C)
        def _(r):
            @pl.loop(0, F, step=16)
            def _(v):
                rows_v[r, pl.ds(v, 16)] = jnp.zeros((16,), jnp.float32)

        @pl.loop(0, STRIPE, step=16)
        def _(i):
            zvec_v[pl.ds(i, 16)] = jnp.zeros((16,), jnp.float32)

        base = sid * STRIPE

        @pl.loop(0, STRIPE, step=C)
        def _(r):
            pltpu.sync_copy(rows_v, acc_sh.at[pl.ds(base + r, C)])

        pltpu.sync_copy(zvec_v, den_sh.at[pl.ds(base, STRIPE)])
        plsc.subcore_barrier()

        @pl.loop(0, NCH)
        def _(j):
            pltpu.sync_copy(h_hbm.at[src_v.at[j]], rows_v)

            for k16 in range(C // 16):
                sl = pl.ds(k16 * 16, 16)
                s16 = src_v[j, sl]
                d16 = dst_v[j, sl]
                eg = plsc.load_gather(es_v, [s16]) + plsc.load_gather(ed_v, [d16])
                eg = jnp.maximum(eg, eg * 0.2)
                ex_v[j, sl] = jnp.exp(eg)

            @pl.loop(0, C)
            def _(c):
                a = ex_v[j, c]
                for v in range(F // 16):
                    sl = pl.ds(v * 16, 16)
                    rows_v[c, sl] = rows_v[c, sl] * a

            pltpu.sync_copy(rows_v, acc_sh.at[dst_v.at[j]], add=True)
            pltpu.sync_copy(ex_v.at[j], den_sh.at[dst_v.at[j]], add=True)

        plsc.subcore_barrier()

        pltpu.sync_copy(acc_sh.at[pl.ds(base, STRIPE)],
                        acc_hbm.at[cid, pl.ds(base, STRIPE)])
        pltpu.sync_copy(den_sh.at[pl.ds(base, STRIPE)],
                        den_hbm.at[cid, pl.ds(base, STRIPE)])

    return k(h, esd, src3, dst3)


def kernel(x, W1, a1_src, a1_dst, W2, a2_src, a2_dst, edge_index):
    ei = edge_index.astype(jnp.int32)
    src3 = ei[0].reshape(NW, NCH, C)
    dst3 = ei[1].reshape(NW, NCH, C)
    a1 = jnp.stack([a1_src, a1_dst])
    a2 = jnp.stack([a2_src, a2_dst])

    h1, esd1p = _dense_in(x, W1, a1)
    esd1 = esd1p[:, :2].T
    acc1, den1 = _edge_sc(h1, esd1, src3, dst3)

    h2, esd2p = _dense_mid(acc1, den1[:, :, None], W2, a2)
    esd2 = esd2p[:, :2].T
    acc2, den2 = _edge_sc(h2, esd2, src3, dst3)

    return _dense_out(acc2, den2[:, :, None])


# speedup vs baseline: 20.6741x; 20.6741x over previous
import dataclasses
import functools

import jax
import jax.numpy as jnp
from jax import lax
from jax.experimental import pallas as pl
from jax.experimental.pallas import tpu as pltpu
from jax.experimental.pallas import tpu_sc as plsc

N = 10000
F = 128
E = 320000
NC = 2
NS = 16
NW = NC * NS
EPW = E // NW
C = 80
NCH = EPW // C
NPAD = 10240
STRIPE = NPAD // NS


def _sc_compiler_params():
    cp = pltpu.CompilerParams()
    if "needs_layout_passes" in pltpu.CompilerParams.__dataclass_fields__:
        cp = dataclasses.replace(cp, needs_layout_passes=False)
    return cp


_SC_MESH = dict(core_axis_name="c", subcore_axis_name="s")


def _dense_in_body(x_ref, w_ref, a_ref, h_ref, esd_ref):
    h = jnp.dot(x_ref[...], w_ref[...], preferred_element_type=jnp.float32,
                precision=lax.Precision.HIGHEST)
    h_ref[...] = h
    es = jnp.sum(h * a_ref[0][None, :], axis=1, keepdims=True)
    ed = jnp.sum(h * a_ref[1][None, :], axis=1, keepdims=True)
    esd_ref[...] = jnp.concatenate(
        [es, ed, jnp.zeros((N, F - 2), jnp.float32)], axis=1)


def _dense_in(x, W, a2):
    return pl.pallas_call(
        _dense_in_body,
        out_shape=[jax.ShapeDtypeStruct((N, F), jnp.float32),
                   jax.ShapeDtypeStruct((N, F), jnp.float32)],
    )(x, W, a2)


def _dense_mid_body(acc_ref, den_ref, w_ref, a_ref, h_ref, esd_ref):
    d = den_ref[0, :N] + den_ref[1, :N]
    y = (acc_ref[0, :N] + acc_ref[1, :N]) / (d + 1e-16)
    y = jnp.maximum(y, 0.0)
    h = jnp.dot(y, w_ref[...], preferred_element_type=jnp.float32,
                precision=lax.Precision.HIGHEST)
    h_ref[...] = h
    es = jnp.sum(h * a_ref[0][None, :], axis=1, keepdims=True)
    ed = jnp.sum(h * a_ref[1][None, :], axis=1, keepdims=True)
    esd_ref[...] = jnp.concatenate(
        [es, ed, jnp.zeros((N, F - 2), jnp.float32)], axis=1)


def _dense_mid(acc, den, W, a2):
    return pl.pallas_call(
        _dense_mid_body,
        out_shape=[jax.ShapeDtypeStruct((N, F), jnp.float32),
                   jax.ShapeDtypeStruct((N, F), jnp.float32)],
    )(acc, den, W, a2)


def _dense_out_body(acc_ref, den_ref, o_ref):
    d = den_ref[0, :N] + den_ref[1, :N]
    o_ref[...] = (acc_ref[0, :N] + acc_ref[1, :N]) / (d + 1e-16)


def _dense_out(acc, den):
    return pl.pallas_call(
        _dense_out_body,
        out_shape=jax.ShapeDtypeStruct((N, F), jnp.float32),
    )(acc, den)


def _edge_weights_sc(esd, src3, dst3):
    mesh = plsc.VectorSubcoreMesh(**_SC_MESH)

    @functools.partial(
        pl.kernel,
        out_type=[jax.ShapeDtypeStruct((NW, NCH, C), jnp.float32),
                  jax.ShapeDtypeStruct((NC, NPAD), jnp.float32)],
        mesh=mesh,
        scratch_types=[
            pltpu.VMEM((N,), jnp.float32),
            pltpu.VMEM((N,), jnp.float32),
            pltpu.VMEM((NCH, C), jnp.int32),
            pltpu.VMEM((NCH, C), jnp.int32),
            pltpu.VMEM((NCH, C), jnp.float32),
            pltpu.VMEM((STRIPE,), jnp.float32),
            pltpu.VMEM_SHARED((NPAD,), jnp.float32),
        ],
        compiler_params=_sc_compiler_params(),
    )
    def k(esd_hbm, src_hbm, dst_hbm, ex_hbm, den_hbm,
          es_v, ed_v, src_v, dst_v, ex_v, zvec_v, den_sh):
        cid = lax.axis_index("c")
        sid = lax.axis_index("s")
        wid = sid * NC + cid

        pltpu.sync_copy(esd_hbm.at[0], es_v)
        pltpu.sync_copy(esd_hbm.at[1], ed_v)
        pltpu.sync_copy(src_hbm.at[wid], src_v)
        pltpu.sync_copy(dst_hbm.at[wid], dst_v)

        @pl.loop(0, STRIPE, step=16)
        def _(i):
            zvec_v[pl.ds(i, 16)] = jnp.zeros((16,), jnp.float32)

        base = sid * STRIPE
        pltpu.sync_copy(zvec_v, den_sh.at[pl.ds(base, STRIPE)])
        plsc.subcore_barrier()

        @pl.loop(0, NCH)
        def _(j):
            @pl.loop(0, C, step=16)
            def _(s):
                sl = pl.ds(s, 16)
                s16 = src_v[j, sl]
                d16 = dst_v[j, sl]
                eg = (plsc.load_gather(es_v, [s16])
                      + plsc.load_gather(ed_v, [d16]))
                eg = jnp.maximum(eg, eg * 0.2)
                ex_v[j, sl] = jnp.exp(eg)

            pltpu.sync_copy(ex_v.at[j], den_sh.at[dst_v.at[j]], add=True)

        pltpu.sync_copy(ex_v, ex_hbm.at[wid])
        plsc.subcore_barrier()
        pltpu.sync_copy(den_sh.at[pl.ds(base, STRIPE)],
                        den_hbm.at[cid, pl.ds(base, STRIPE)])

    return k(esd, src3, dst3)


def _edge_rows_sc(h, src3, dst3, ex3):
    mesh = plsc.VectorSubcoreMesh(**_SC_MESH)

    @functools.partial(
        pl.kernel,
        out_type=jax.ShapeDtypeStruct((NC, NPAD, F), jnp.float32),
        mesh=mesh,
        scratch_types=[
            pltpu.VMEM((1, C), jnp.int32),
            pltpu.VMEM((1, C), jnp.int32),
            pltpu.VMEM((1, C), jnp.float32),
            pltpu.VMEM((C, F), jnp.float32),
            pltpu.VMEM_SHARED((NPAD, F), jnp.float32),
        ],
        compiler_params=_sc_compiler_params(),
    )
    def k(h_hbm, src_hbm, dst_hbm, ex_hbm, acc_hbm,
          src_v, dst_v, ex_v, rows_v, acc_sh):
        cid = lax.axis_index("c")
        sid = lax.axis_index("s")
        wid = sid * NC + cid

        @pl.loop(0, C)
        def _(r):
            @pl.loop(0, F, step=16)
            def _(v):
                rows_v[r, pl.ds(v, 16)] = jnp.zeros((16,), jnp.float32)

        base = sid * STRIPE

        @pl.loop(0, STRIPE, step=C)
        def _(r):
            pltpu.sync_copy(rows_v, acc_sh.at[pl.ds(base + r, C)])

        plsc.subcore_barrier()

        @pl.loop(0, NCH)
        def _(j):
            pltpu.sync_copy(src_hbm.at[wid, j], src_v.at[0])
            pltpu.sync_copy(dst_hbm.at[wid, j], dst_v.at[0])
            pltpu.sync_copy(ex_hbm.at[wid, j], ex_v.at[0])
            pltpu.sync_copy(h_hbm.at[src_v.at[0]], rows_v)

            @pl.loop(0, C, step=16)
            def _(s):
                ex16 = ex_v[0, pl.ds(s, 16)]
                for l in range(16):
                    a = ex16[l]
                    for v in range(F // 16):
                        slv = pl.ds(v * 16, 16)
                        rows_v[s + l, slv] = rows_v[s + l, slv] * a

            pltpu.sync_copy(rows_v, acc_sh.at[dst_v.at[0]], add=True)

        plsc.subcore_barrier()
        pltpu.sync_copy(acc_sh.at[pl.ds(base, STRIPE)],
                        acc_hbm.at[cid, pl.ds(base, STRIPE)])

    return k(h, src3, dst3, ex3)


def kernel(x, W1, a1_src, a1_dst, W2, a2_src, a2_dst, edge_index):
    ei = edge_index.astype(jnp.int32)
    src3 = ei[0].reshape(NW, NCH, C)
    dst3 = ei[1].reshape(NW, NCH, C)
    a1 = jnp.stack([a1_src, a1_dst])
    a2 = jnp.stack([a2_src, a2_dst])

    h1, esd1p = _dense_in(x, W1, a1)
    esd1 = esd1p[:, :2].T
    ex1, den1 = _edge_weights_sc(esd1, src3, dst3)
    acc1 = _edge_rows_sc(h1, src3, dst3, ex1)

    h2, esd2p = _dense_mid(acc1, den1[:, :, None], W2, a2)
    esd2 = esd2p[:, :2].T
    ex2, den2 = _edge_weights_sc(esd2, src3, dst3)
    acc2 = _edge_rows_sc(h2, src3, dst3, ex2)

    return _dense_out(acc2, den2[:, :, None])
